# Initial kernel scaffold; baseline (speedup 1.0000x reference)
import jax
import jax.numpy as jnp
from jax.experimental import pallas as pl


def kernel(hidden_states, gate_w, e_bias, w_gate, w_up, w_down):
    raise NotImplementedError("write your pallas kernel here")


# speedup vs baseline: 2.0513x; 2.0513x over previous
import functools

import jax
import jax.numpy as jnp
from jax.experimental import pallas as pl
from jax.experimental.pallas import tpu as pltpu

E = 8
TOP_K = 2
N_GROUP = 4
TOPK_GROUP = 2
D_MODEL = 1024
D_FF = 768
T = 2048

_NEG = -1e30


def _topk_mask_cols(cols, k):
    n = len(cols)
    masks = []
    for e in range(n):
        rank = jnp.zeros_like(cols[0], dtype=jnp.int32)
        for j in range(n):
            if j == e:
                continue
            beats = cols[j] > cols[e]
            if j < e:
                beats = beats | (cols[j] == cols[e])
            rank = rank + beats.astype(jnp.int32)
        masks.append((rank < k).astype(jnp.float32))
    return masks


def _compute_combine(x, gate_w, e_bias):
    logits = jax.lax.dot_general(
        x, gate_w, (((1,), (1,)), ((), ())),
        preferred_element_type=jnp.float32)
    scores = 1.0 / (1.0 + jnp.exp(-logits))
    sfc = scores + e_bias
    sfc_cols = [sfc[:, j:j + 1] for j in range(E)]
    gsz = E // N_GROUP
    g_cols = []
    for g in range(N_GROUP):
        s = sfc_cols[g * gsz]
        for i in range(1, gsz):
            s = s + sfc_cols[g * gsz + i]
        g_cols.append(s)
    g_masks = _topk_mask_cols(g_cols, TOPK_GROUP)
    masked_cols = []
    for e in range(E):
        gm = g_masks[e // gsz]
        masked_cols.append(jnp.where(gm > 0.0, sfc_cols[e], _NEG))
    sel = _topk_mask_cols(masked_cols, TOP_K)
    sel2 = jnp.concatenate(sel, axis=1)
    w_raw = sel2 * scores
    denom = jnp.sum(w_raw, axis=1, keepdims=True) + 1e-20
    return w_raw / denom


def _router_kernel(x_ref, gw_ref, eb_ref, combine_ref):
    combine_ref[...] = _compute_combine(x_ref[...], gw_ref[...], eb_ref[...])


_RB = 256


def _router(x, gate_w, e_bias):
    return pl.pallas_call(
        _router_kernel,
        grid=(T // _RB,),
        in_specs=[
            pl.BlockSpec((_RB, D_MODEL), lambda i: (i, 0)),
            pl.BlockSpec((E, D_MODEL), lambda i: (0, 0)),
            pl.BlockSpec((1, E), lambda i: (0, 0)),
        ],
        out_specs=pl.BlockSpec((_RB, E), lambda i: (i, 0)),
        out_shape=jax.ShapeDtypeStruct((T, E), jnp.float32),
        compiler_params=pltpu.CompilerParams(
            dimension_semantics=("arbitrary",),
        ),
    )(x, gate_w, e_bias.reshape(1, E))


_TC = 1024


def _moe_kernel(x_ref, combine_in_ref, wg_ref, wu_ref, wd_ref, out_ref):
    e = pl.program_id(0)
    combine_ref = combine_in_ref
    wg = wg_ref[0]
    wu = wu_ref[0]
    wd = wd_ref[0]
    lane = jax.lax.broadcasted_iota(jnp.int32, (_TC, E), 1)

    def chunk(c, _):
        off = pl.multiple_of(c * _TC, _TC)
        x = x_ref[pl.ds(off, _TC), :]
        g = jnp.dot(x, wg, preferred_element_type=jnp.float32)
        u = jnp.dot(x, wu, preferred_element_type=jnp.float32)
        h = (g / (1.0 + jnp.exp(-g))) * u
        y = jnp.dot(h, wd, preferred_element_type=jnp.float32)
        c_e = jnp.sum(
            jnp.where(lane == e, combine_ref[pl.ds(off, _TC), :], 0.0),
            axis=1, keepdims=True)
        contrib = c_e * y

        @pl.when(e == 0)
        def _():
            out_ref[pl.ds(off, _TC), :] = contrib

        @pl.when(e > 0)
        def _():
            out_ref[pl.ds(off, _TC), :] = out_ref[pl.ds(off, _TC), :] + contrib

        return 0

    jax.lax.fori_loop(0, T // _TC, chunk, 0)


@jax.jit
def kernel(hidden_states, gate_w, e_bias, w_gate, w_up, w_down):
    x = hidden_states.reshape(-1, D_MODEL)
    combine = _router(x, gate_w, e_bias)
    out = pl.pallas_call(
        _moe_kernel,
        grid=(E,),
        in_specs=[
            pl.BlockSpec((T, D_MODEL), lambda e: (0, 0)),
            pl.BlockSpec((T, E), lambda e: (0, 0)),
            pl.BlockSpec((1, D_MODEL, D_FF), lambda e: (e, 0, 0)),
            pl.BlockSpec((1, D_MODEL, D_FF), lambda e: (e, 0, 0)),
            pl.BlockSpec((1, D_FF, D_MODEL), lambda e: (e, 0, 0)),
        ],
        out_specs=pl.BlockSpec((T, D_MODEL), lambda e: (0, 0)),
        out_shape=jax.ShapeDtypeStruct((T, D_MODEL), jnp.float32),
        compiler_params=pltpu.CompilerParams(
            dimension_semantics=("arbitrary",),
        ),
    )(x, combine, w_gate, w_up, w_down)
    return out
